# Initial kernel scaffold; baseline (speedup 1.0000x reference)
import jax
import jax.numpy as jnp
from jax.experimental import pallas as pl


def kernel(A_batch, env_labels):
    raise NotImplementedError("write your pallas kernel here")


# speedup vs baseline: 1.8200x; 1.8200x over previous
import jax
import jax.numpy as jnp
from jax.experimental import pallas as pl
from jax.experimental.pallas import tpu as pltpu

_PENALTY_WEIGHT = 1.0
_MIN_ENV_SAMPLES = 2.0
_E = 8


def _irm_kernel(lab_ref, a_ref, out_ref):
    i = pl.program_id(0)
    n = pl.num_programs(0)
    labs = lab_ref[0, :]
    oh = (labs[None, :] == jax.lax.broadcasted_iota(
        jnp.int32, (_E, labs.shape[0]), 0)).astype(jnp.float32)
    counts = jnp.sum(oh, axis=1)
    sums = jnp.dot(oh, a_ref[...], preferred_element_type=jnp.float32)
    valid = (counts >= _MIN_ENV_SAMPLES).astype(jnp.float32)
    safe = jnp.maximum(counts, 1.0)
    means = sums / safe[:, None]
    n_valid = jnp.sum(valid)
    w = valid[:, None]
    mom = jnp.sum(means * w, axis=0) / n_valid
    var = jnp.sum(w * (means - mom[None, :]) ** 2, axis=0) / (n_valid - 1.0)
    part = jnp.sum(var)

    @pl.when(i == 0)
    def _init():
        out_ref[0, 0] = 0.0

    out_ref[0, 0] += part


def kernel(A_batch, env_labels):
    b, d, _ = A_batch.shape
    a_flat = A_batch.reshape(b, d * d)
    labs = env_labels.astype(jnp.int32).reshape(1, b)
    bn = 2048
    grid = (d * d // bn,)
    out = pl.pallas_call(
        _irm_kernel,
        grid=grid,
        in_specs=[
            pl.BlockSpec((1, b), lambda i: (0, 0)),
            pl.BlockSpec((b, bn), lambda i: (0, i)),
        ],
        out_specs=pl.BlockSpec((1, 1), lambda i: (0, 0),
                               memory_space=pltpu.SMEM),
        out_shape=jax.ShapeDtypeStruct((1, 1), jnp.float32),
        compiler_params=pltpu.CompilerParams(
            dimension_semantics=("arbitrary",),
        ),
    )(labs, a_flat)
    return out[0, 0] * (_PENALTY_WEIGHT / (d * d))
